# baseline (speedup 1.0000x reference)
import jax
import jax.numpy as jnp
from jax.experimental import pallas as pl
from jax.experimental.pallas import tpu as pltpu

B = 256
IN_DIM = 512
OUT_DIM = 10000
BJ = 400
NJ = OUT_DIM // BJ


def _gcn_kernel(wT_ref, xT_ref, adj_ref, bias_ref, out_ref):
    t = jnp.dot(
        adj_ref[...].astype(jnp.bfloat16),
        wT_ref[...],
        preferred_element_type=jnp.float32,
    )
    out_ref[...] = (
        jnp.dot(t.astype(jnp.bfloat16), xT_ref[...], preferred_element_type=jnp.float32)
        + bias_ref[...]
    )


def kernel(input, adj, weight, bias):
    wT = weight.T.astype(jnp.bfloat16)
    xT = input.T.astype(jnp.bfloat16)
    outT = pl.pallas_call(
        _gcn_kernel,
        grid=(NJ,),
        in_specs=[
            pl.BlockSpec((OUT_DIM, IN_DIM), lambda j: (0, 0)),
            pl.BlockSpec((IN_DIM, B), lambda j: (0, 0)),
            pl.BlockSpec((BJ, OUT_DIM), lambda j: (j, 0)),
            pl.BlockSpec((BJ, 1), lambda j: (j, 0)),
        ],
        out_specs=pl.BlockSpec((BJ, B), lambda j: (j, 0)),
        out_shape=jax.ShapeDtypeStruct((OUT_DIM, B), jnp.float32),
        compiler_params=pltpu.CompilerParams(
            dimension_semantics=("parallel",),
        ),
    )(wT, xT, adj, bias.reshape(OUT_DIM, 1))
    return outT.T


# speedup vs baseline: 1.0504x; 1.0504x over previous
import jax
import jax.numpy as jnp
from jax.experimental import pallas as pl
from jax.experimental.pallas import tpu as pltpu

B = 256
IN_DIM = 512
OUT_DIM = 10000
BJ = 400
NJ = OUT_DIM // BJ


def _gcn_kernel(wT_ref, xT_ref, adj_ref, bias_ref, out_ref, sT_ref):
    @pl.when(pl.program_id(0) == 0)
    def _():
        sT_ref[...] = jnp.dot(
            wT_ref[...], xT_ref[...], preferred_element_type=jnp.float32
        ).astype(jnp.bfloat16)

    out_ref[...] = (
        jnp.dot(
            adj_ref[...].astype(jnp.bfloat16),
            sT_ref[...],
            preferred_element_type=jnp.float32,
        )
        + bias_ref[...]
    )


def kernel(input, adj, weight, bias):
    wT = weight.T.astype(jnp.bfloat16)
    xT = input.T.astype(jnp.bfloat16)
    outT = pl.pallas_call(
        _gcn_kernel,
        grid=(NJ,),
        in_specs=[
            pl.BlockSpec((OUT_DIM, IN_DIM), lambda j: (0, 0)),
            pl.BlockSpec((IN_DIM, B), lambda j: (0, 0)),
            pl.BlockSpec((BJ, OUT_DIM), lambda j: (j, 0)),
            pl.BlockSpec((BJ, 1), lambda j: (j, 0)),
        ],
        out_specs=pl.BlockSpec((BJ, B), lambda j: (j, 0)),
        out_shape=jax.ShapeDtypeStruct((OUT_DIM, B), jnp.float32),
        scratch_shapes=[pltpu.VMEM((OUT_DIM, B), jnp.bfloat16)],
    )(wT, xT, adj, bias.reshape(OUT_DIM, 1))
    return outT.T
